# baseline (speedup 1.0000x reference)
import functools

import jax
import jax.numpy as jnp
from jax import lax
from jax.experimental import pallas as pl
from jax.experimental.pallas import tpu as pltpu
from jax.experimental.pallas import tpu_sc as plsc

_TARGET_LEN = 2048


def _gather_rows_sc(table, seq_len, num_rows, dim, rows_per_w, nbuf):
    info = plsc.get_sparse_core_info()
    nc, ns = info.num_cores, info.num_subcores
    chunk = info.num_lanes
    n_ch = rows_per_w // chunk

    mesh = plsc.VectorSubcoreMesh(core_axis_name="c", subcore_axis_name="s")

    @functools.partial(
        pl.kernel,
        out_type=jax.ShapeDtypeStruct((num_rows, dim), jnp.float32),
        mesh=mesh,
        scratch_types=[
            pltpu.VMEM((nbuf, chunk, dim), jnp.float32),
            pltpu.SemaphoreType.DMA((nbuf,)),
            pltpu.SemaphoreType.DMA((nbuf,)),
        ],
    )
    def body(table_hbm, out_hbm, buf_v, in_sems, out_sems):
        wid = lax.axis_index("c") * ns + lax.axis_index("s")
        base = wid * rows_per_w

        in_d = [None] * nbuf
        out_d = [None] * nbuf

        def flat_idx_vec(g):
            j = base + g * chunk + lax.iota(jnp.int32, chunk)
            t_log2 = _TARGET_LEN.bit_length() - 1
            q = (seq_len - 1) // (_TARGET_LEN - 1)
            r = (seq_len - 1) - q * (_TARGET_LEN - 1)
            jj = j & (_TARGET_LEN - 1)
            flat = (j >> t_log2) * seq_len + jj * q + r
            for k in range(1, r + 1):
                t_k = -((-k * (_TARGET_LEN - 1)) // r)
                flat = flat - lax.shift_right_logical(jj - t_k, 31)
            return flat

        def issue_gather(g):
            slot = g % nbuf
            in_d[slot] = pltpu.async_copy(
                table_hbm.at[flat_idx_vec(g)], buf_v.at[slot], in_sems.at[slot]
            )

        for g in range(min(nbuf - 1, n_ch)):
            issue_gather(g)
        for g in range(n_ch):
            slot = g % nbuf
            in_d[slot].wait()
            out_d[slot] = pltpu.async_copy(
                buf_v.at[slot],
                out_hbm.at[pl.ds(base + g * chunk, chunk)],
                out_sems.at[slot],
            )
            nxt = g + nbuf - 1
            if nxt < n_ch:
                nslot = nxt % nbuf
                if out_d[nslot] is not None:
                    out_d[nslot].wait()
                issue_gather(nxt)
        for k in range(max(0, n_ch - nbuf), n_ch):
            out_d[k % nbuf].wait()

    return body(table)


def kernel(output_tokens):
    batch, seq_len, dim = output_tokens.shape
    table = output_tokens.reshape(batch * seq_len, dim)

    num_rows = batch * _TARGET_LEN
    rows_per_w = num_rows // 32
    nbuf = 5

    out = _gather_rows_sc(table, seq_len, num_rows, dim, rows_per_w, nbuf)
    return out.reshape(batch, _TARGET_LEN, dim)


# speedup vs baseline: 1.0145x; 1.0145x over previous
import functools

import jax
import jax.numpy as jnp
from jax import lax
from jax.experimental import pallas as pl
from jax.experimental.pallas import tpu as pltpu
from jax.experimental.pallas import tpu_sc as plsc

_TARGET_LEN = 2048


def _gather_rows_sc(table, seq_len, num_rows, dim, rows_per_w, nbuf):
    info = plsc.get_sparse_core_info()
    nc, ns = info.num_cores, info.num_subcores
    chunk = info.num_lanes
    n_ch = rows_per_w // chunk

    mesh = plsc.VectorSubcoreMesh(core_axis_name="c", subcore_axis_name="s")

    @functools.partial(
        pl.kernel,
        out_type=jax.ShapeDtypeStruct((num_rows, dim), jnp.float32),
        mesh=mesh,
        scratch_types=[
            pltpu.VMEM((nbuf, chunk, dim), jnp.float32),
            pltpu.SemaphoreType.DMA((nbuf,)),
            pltpu.SemaphoreType.DMA((nbuf,)),
        ],
    )
    def body(table_hbm, out_hbm, buf_v, in_sems, out_sems):
        wid = lax.axis_index("c") * ns + lax.axis_index("s")
        base = wid * rows_per_w

        in_d = [None] * nbuf
        out_d = [None] * nbuf

        def flat_idx_vec(g):
            j = base + g * chunk + lax.iota(jnp.int32, chunk)
            t_log2 = _TARGET_LEN.bit_length() - 1
            q = (seq_len - 1) // (_TARGET_LEN - 1)
            r = (seq_len - 1) - q * (_TARGET_LEN - 1)
            jj = j & (_TARGET_LEN - 1)
            flat = (j >> t_log2) * seq_len + jj * q + r
            for k in range(1, r + 1):
                t_k = -((-k * (_TARGET_LEN - 1)) // r)
                flat = flat - lax.shift_right_logical(jj - t_k, 31)
            return flat

        def issue_gather(g):
            slot = g % nbuf
            in_d[slot] = pltpu.async_copy(
                table_hbm.at[flat_idx_vec(g)], buf_v.at[slot], in_sems.at[slot]
            )

        for g in range(min(nbuf - 1, n_ch)):
            issue_gather(g)
        for g in range(n_ch):
            slot = g % nbuf
            in_d[slot].wait()
            out_d[slot] = pltpu.async_copy(
                buf_v.at[slot],
                out_hbm.at[pl.ds(base + g * chunk, chunk)],
                out_sems.at[slot],
            )
            nxt = g + nbuf - 1
            if nxt < n_ch:
                nslot = nxt % nbuf
                if out_d[nslot] is not None:
                    out_d[nslot].wait()
                issue_gather(nxt)
        for k in range(max(0, n_ch - nbuf), n_ch):
            out_d[k % nbuf].wait()

    return body(table)


def kernel(output_tokens):
    batch, seq_len, dim = output_tokens.shape
    table = output_tokens.reshape(batch * seq_len, dim)

    num_rows = batch * _TARGET_LEN
    rows_per_w = num_rows // 32
    nbuf = 6

    out = _gather_rows_sc(table, seq_len, num_rows, dim, rows_per_w, nbuf)
    return out.reshape(batch, _TARGET_LEN, dim)
